# baseline (speedup 1.0000x reference)
import functools

import jax
import jax.numpy as jnp
from jax import lax
from jax.experimental import pallas as pl
from jax.experimental.pallas import tpu as pltpu
from jax.experimental.pallas import tpu_sc as plsc

EMBED = 32
ROW_SCALE = 32.0 ** 0.5
NC, NS = 2, 16
NW = NC * NS
SEG = 128
SB = 16
NBUF = 8
RU = 8


def _emb_call(n_seg):
    segs_per_w = n_seg // NW
    n_batches = segs_per_w // SB
    assert segs_per_w * NW == n_seg and n_batches * SB == segs_per_w
    mesh = plsc.VectorSubcoreMesh(core_axis_name="c", subcore_axis_name="s",
                                  num_cores=NC, num_subcores=NS)

    @functools.partial(
        pl.kernel,
        out_type=jax.ShapeDtypeStruct((n_seg // 128, 4, 128, 8, SEG),
                                      jnp.float32),
        mesh=mesh,
        compiler_params=pltpu.CompilerParams(use_tc_tiling_on_sc=False,
                                             needs_layout_passes=False),
        scratch_types=[
            pltpu.VMEM((2, SB, SEG), jnp.int32),
            pltpu.VMEM((NBUF, SEG, EMBED), jnp.float32),
        ] + [pltpu.VMEM((4, 8, SEG + 1), jnp.float32)] * NBUF
          + [pltpu.SemaphoreType.DMA] * (2 * NBUF + 2),
    )
    def body(idx_hbm, table_hbm, out_hbm, idx_v, rows_v, *rest):
        t_refs, sems = rest[:NBUF], rest[NBUF:]
        sg, st, si = sems[:NBUF], sems[NBUF:2 * NBUF], sems[2 * NBUF:]
        wid = lax.axis_index("s") * NC + lax.axis_index("c")
        sbase = wid * segs_per_w

        def fire_gather(r, ib, row):
            pltpu.async_copy(table_hbm.at[idx_v.at[ib, row]],
                             rows_v.at[r], sg[r])

        def wait_gather(r):
            pltpu.make_async_copy(table_hbm.at[pl.ds(0, SEG)],
                                  rows_v.at[r], sg[r]).wait()

        def wait_stores(r):
            pltpu.make_async_copy(
                out_hbm.at[0, pl.ds(0, 4), 0],
                t_refs[r].at[pl.ds(0, 4), pl.ds(0, 8), pl.ds(0, SEG)],
                st[r]).wait()

        def fire_idx(k, ib):
            row0 = pl.multiple_of(sbase + k * SB, SB)
            pltpu.async_copy(idx_hbm.at[pl.ds(row0, SB)],
                             idx_v.at[ib], si[ib])

        def wait_idx(ib):
            pltpu.make_async_copy(idx_hbm.at[pl.ds(0, SB)],
                                  idx_v.at[ib], si[ib]).wait()

        lane = lax.iota(jnp.int32, 16)
        zero16 = jnp.zeros((16,), jnp.int32)

        m_lo = lane >> 3
        m_hi = m_lo + 2
        r_ln = lane & 7

        def transpose_scale(r):
            @plsc.parallel_loop(0, SEG, step=RU)
            def _(c0):
                cvec = zero16 + c0
                for u in range(RU):
                    v0 = rows_v[r, c0 + u, pl.ds(0, 16)] * ROW_SCALE
                    v1 = rows_v[r, c0 + u, pl.ds(16, 16)] * ROW_SCALE
                    plsc.store_scatter(t_refs[r], [m_lo, r_ln, cvec + u], v0)
                    plsc.store_scatter(t_refs[r], [m_hi, r_ln, cvec + u], v1)

        def fire_stores(s, r):
            jt = s >> 10
            it = (s >> 3) & 127
            jr = s & 7
            j = jt * 8 + jr
            pltpu.async_copy(
                t_refs[r].at[pl.ds(0, 4), pl.ds(0, 8), pl.ds(0, SEG)],
                out_hbm.at[j, pl.ds(0, 4), it], st[r])

        fire_idx(0, 0)
        wait_idx(0)
        for r in range(7):
            fire_gather(r, 0, r)

        def outer(gg, carry):
            for kk in range(2):
                k = gg * 2 + kk
                ib = kk

                @pl.when(k + 1 < n_batches)
                def _():
                    fire_idx(k + 1, 1 - ib)

                for bi in range(SB):
                    s = k * SB + bi
                    r = bi % NBUF
                    r7 = (bi + 7) % NBUF
                    wait_gather(r)

                    if bi == 9:
                        @pl.when(k + 1 < n_batches)
                        def _():
                            wait_idx(1 - ib)

                    @pl.when(s + 7 < segs_per_w)
                    def _():
                        if bi < SB - 7:
                            fire_gather(r7, ib, bi + 7)
                        else:
                            fire_gather(r7, 1 - ib, bi + 7 - SB)

                    @pl.when(s >= NBUF)
                    def _():
                        wait_stores(r)

                    transpose_scale(r)
                    fire_stores(sbase + s, r)
            return carry

        lax.fori_loop(0, n_batches // 2, outer, 0)
        for r in range(NBUF):
            wait_stores(r)

    return body


def kernel(x, table):
    s0, s1 = x.shape
    n_seg = (s0 // SEG) * s1
    x5 = (x.astype(jnp.int32)
           .reshape(s0 // SEG, SEG, s1 // 8, 8)
           .transpose(2, 0, 3, 1)
           .reshape(n_seg, SEG))
    o6 = _emb_call(n_seg)(x5, table.astype(jnp.float32))
    return o6.transpose(2, 4, 0, 1, 3).reshape(s0, s1, EMBED)


# speedup vs baseline: 1.0317x; 1.0317x over previous
import functools

import jax
import jax.numpy as jnp
from jax import lax
from jax.experimental import pallas as pl
from jax.experimental.pallas import tpu as pltpu
from jax.experimental.pallas import tpu_sc as plsc

EMBED = 32
ROW_SCALE = 32.0 ** 0.5
NC, NS = 2, 16
NW = NC * NS
SEG = 128
SB = 8
NBUF = 8
RU = 8


def _emb_call(n_seg):
    segs_per_w = n_seg // NW
    n_batches = segs_per_w // SB
    assert segs_per_w * NW == n_seg and n_batches * SB == segs_per_w
    mesh = plsc.VectorSubcoreMesh(core_axis_name="c", subcore_axis_name="s",
                                  num_cores=NC, num_subcores=NS)

    @functools.partial(
        pl.kernel,
        out_type=jax.ShapeDtypeStruct((n_seg // 128, 4, 128, 8, SEG),
                                      jnp.float32),
        mesh=mesh,
        compiler_params=pltpu.CompilerParams(use_tc_tiling_on_sc=False,
                                             needs_layout_passes=False),
        scratch_types=[
            pltpu.VMEM((2, SB, SEG), jnp.int32),
            pltpu.VMEM((NBUF, SEG, EMBED), jnp.float32),
        ] + [pltpu.VMEM((4, 8, SEG + 1), jnp.float32)] * NBUF
          + [pltpu.SemaphoreType.DMA] * (2 * NBUF + 2),
    )
    def body(idx_hbm, table_hbm, out_hbm, idx_v, rows_v, *rest):
        t_refs, sems = rest[:NBUF], rest[NBUF:]
        sg, st, si = sems[:NBUF], sems[NBUF:2 * NBUF], sems[2 * NBUF:]
        wid = lax.axis_index("s") * NC + lax.axis_index("c")
        sbase = wid * segs_per_w

        def fire_gather(r, ib, row):
            pltpu.async_copy(table_hbm.at[idx_v.at[ib, row]],
                             rows_v.at[r], sg[r])

        def wait_gather(r):
            pltpu.make_async_copy(table_hbm.at[pl.ds(0, SEG)],
                                  rows_v.at[r], sg[r]).wait()

        def wait_stores(r):
            pltpu.make_async_copy(
                out_hbm.at[0, pl.ds(0, 4), 0],
                t_refs[r].at[pl.ds(0, 4), pl.ds(0, 8), pl.ds(0, SEG)],
                st[r]).wait()

        def fire_idx(k, ib):
            row0 = pl.multiple_of(sbase + k * SB, SB)
            pltpu.async_copy(idx_hbm.at[pl.ds(row0, SB)],
                             idx_v.at[ib], si[ib])

        def wait_idx(ib):
            pltpu.make_async_copy(idx_hbm.at[pl.ds(0, SB)],
                                  idx_v.at[ib], si[ib]).wait()

        lane = lax.iota(jnp.int32, 16)
        zero16 = jnp.zeros((16,), jnp.int32)

        m_lo = lane >> 3
        m_hi = m_lo + 2
        r_ln = lane & 7

        def transpose_scale(r):
            @plsc.parallel_loop(0, SEG, step=RU)
            def _(c0):
                cvec = zero16 + c0
                for u in range(RU):
                    v0 = rows_v[r, c0 + u, pl.ds(0, 16)] * ROW_SCALE
                    v1 = rows_v[r, c0 + u, pl.ds(16, 16)] * ROW_SCALE
                    plsc.store_scatter(t_refs[r], [m_lo, r_ln, cvec + u], v0)
                    plsc.store_scatter(t_refs[r], [m_hi, r_ln, cvec + u], v1)

        def fire_stores(s, r):
            jt = s >> 10
            it = (s >> 3) & 127
            jr = s & 7
            j = jt * 8 + jr
            pltpu.async_copy(
                t_refs[r].at[pl.ds(0, 4), pl.ds(0, 8), pl.ds(0, SEG)],
                out_hbm.at[j, pl.ds(0, 4), it], st[r])

        fire_idx(0, 0)
        wait_idx(0)
        for r in range(5):
            fire_gather(r, 0, r)

        def outer(gg, carry):
            for kk in range(2):
                k = gg * 2 + kk
                ib = kk

                @pl.when(k + 1 < n_batches)
                def _():
                    fire_idx(k + 1, 1 - ib)

                for bi in range(SB):
                    s = k * SB + bi
                    r = bi
                    r5 = (bi + 5) % NBUF
                    wait_gather(r)

                    if bi == 3:
                        @pl.when(k + 1 < n_batches)
                        def _():
                            wait_idx(1 - ib)

                    @pl.when(s + 5 < segs_per_w)
                    def _():
                        if bi < 3:
                            fire_gather(r5, ib, bi + 5)
                        else:
                            fire_gather(r5, 1 - ib, bi - 3)

                    @pl.when(s >= NBUF)
                    def _():
                        wait_stores(r)

                    transpose_scale(r)
                    fire_stores(sbase + s, r)
            return carry

        lax.fori_loop(0, n_batches // 2, outer, 0)
        for r in range(NBUF):
            wait_stores(r)

    return body


def kernel(x, table):
    s0, s1 = x.shape
    n_seg = (s0 // SEG) * s1
    x5 = (x.astype(jnp.int32)
           .reshape(s0 // SEG, SEG, s1 // 8, 8)
           .transpose(2, 0, 3, 1)
           .reshape(n_seg, SEG))
    o6 = _emb_call(n_seg)(x5, table.astype(jnp.float32))
    return o6.transpose(2, 4, 0, 1, 3).reshape(s0, s1, EMBED)


# speedup vs baseline: 1.0435x; 1.0114x over previous
import functools

import jax
import jax.numpy as jnp
from jax import lax
from jax.experimental import pallas as pl
from jax.experimental.pallas import tpu as pltpu
from jax.experimental.pallas import tpu_sc as plsc

EMBED = 32
ROW_SCALE = 32.0 ** 0.5
NC, NS = 2, 16
NW = NC * NS
SEG = 128
SB = 8
NBUF = 8
RU = 8


def _emb_call(n_seg):
    segs_per_w = n_seg // NW
    n_batches = segs_per_w // SB
    assert segs_per_w * NW == n_seg and n_batches * SB == segs_per_w
    mesh = plsc.VectorSubcoreMesh(core_axis_name="c", subcore_axis_name="s",
                                  num_cores=NC, num_subcores=NS)

    @functools.partial(
        pl.kernel,
        out_type=jax.ShapeDtypeStruct((n_seg // 128, 4, 128, 8, SEG),
                                      jnp.float32),
        mesh=mesh,
        compiler_params=pltpu.CompilerParams(use_tc_tiling_on_sc=False,
                                             needs_layout_passes=False),
        scratch_types=[
            pltpu.VMEM((2, SB, SEG), jnp.int32),
            pltpu.VMEM((NBUF, SEG, EMBED), jnp.float32),
        ] + [pltpu.VMEM((4, 8, SEG + 1), jnp.float32)] * NBUF
          + [pltpu.SemaphoreType.DMA] * (2 * NBUF + 2),
    )
    def body(idx_hbm, table_hbm, out_hbm, idx_v, rows_v, *rest):
        t_refs, sems = rest[:NBUF], rest[NBUF:]
        sg, st, si = sems[:NBUF], sems[NBUF:2 * NBUF], sems[2 * NBUF:]
        wid = lax.axis_index("s") * NC + lax.axis_index("c")
        sbase = wid * segs_per_w

        def fire_gather(r, ib, row):
            pltpu.async_copy(table_hbm.at[idx_v.at[ib, row]],
                             rows_v.at[r], sg[r])

        def wait_gather(r):
            pltpu.make_async_copy(table_hbm.at[pl.ds(0, SEG)],
                                  rows_v.at[r], sg[r]).wait()

        def wait_stores(r):
            pltpu.make_async_copy(
                out_hbm.at[0, pl.ds(0, 4), 0],
                t_refs[r].at[pl.ds(0, 4), pl.ds(0, 8), pl.ds(0, SEG)],
                st[r]).wait()

        def fire_idx(k, ib):
            row0 = pl.multiple_of(sbase + k * SB, SB)
            pltpu.async_copy(idx_hbm.at[pl.ds(row0, SB)],
                             idx_v.at[ib], si[ib])

        def wait_idx(ib):
            pltpu.make_async_copy(idx_hbm.at[pl.ds(0, SB)],
                                  idx_v.at[ib], si[ib]).wait()

        lane = lax.iota(jnp.int32, 16)
        zero16 = jnp.zeros((16,), jnp.int32)

        m_lo = lane >> 3
        m_hi = m_lo + 2
        r_ln = lane & 7

        def transpose_scale(r):
            @plsc.parallel_loop(0, SEG, step=RU)
            def _(c0):
                cvec = zero16 + c0
                for u in range(RU):
                    v0 = rows_v[r, c0 + u, pl.ds(0, 16)] * ROW_SCALE
                    v1 = rows_v[r, c0 + u, pl.ds(16, 16)] * ROW_SCALE
                    plsc.store_scatter(t_refs[r], [m_lo, r_ln, cvec + u], v0)
                    plsc.store_scatter(t_refs[r], [m_hi, r_ln, cvec + u], v1)

        def fire_stores(s, r):
            jt = s >> 10
            it = (s >> 3) & 127
            jr = s & 7
            j = jt * 8 + jr
            pltpu.async_copy(
                t_refs[r].at[pl.ds(0, 4), pl.ds(0, 8), pl.ds(0, SEG)],
                out_hbm.at[j, pl.ds(0, 4), it], st[r])

        fire_idx(0, 0)
        wait_idx(0)
        for r in range(5):
            fire_gather(r, 0, r)

        def outer(gg, carry):
            for kk in range(2):
                k = gg * 2 + kk
                ib = kk

                @pl.when(k + 1 < n_batches)
                def _():
                    fire_idx(k + 1, 1 - ib)

                for bi in range(SB):
                    s = k * SB + bi
                    r = bi
                    r5 = (bi + 5) % NBUF
                    wait_gather(r)

                    if bi == 3:
                        @pl.when(k + 1 < n_batches)
                        def _():
                            wait_idx(1 - ib)

                    @pl.when(s + 5 < segs_per_w)
                    def _():
                        if bi < 3:
                            fire_gather(r5, ib, bi + 5)
                        else:
                            fire_gather(r5, 1 - ib, bi - 3)

                    @pl.when(s >= NBUF)
                    def _():
                        wait_stores(r)

                    transpose_scale(r)
                    fire_stores(sbase + s, r)
            return carry

        lax.fori_loop(0, n_batches // 2, outer, 0)
        for r in range(NBUF):
            wait_stores(r)

    return body


def kernel(x, table):
    s0, s1 = x.shape
    n_seg = (s0 // SEG) * s1
    x5 = (x.astype(jnp.int32)
           .reshape(s0 // SEG, SEG, s1 // 8, 8)
           .transpose(2, 0, 3, 1)
           .reshape(n_seg, SEG))
    tpad = jnp.pad(table.astype(jnp.float32), ((0, 0), (0, 96)))
    o6 = _emb_call(n_seg)(x5 * 4, tpad.reshape(-1, EMBED))
    return o6.transpose(2, 4, 0, 1, 3).reshape(s0, s1, EMBED)
